# baseline (speedup 1.0000x reference)
import functools

import jax
import jax.numpy as jnp
from jax import lax
from jax.experimental import pallas as pl
from jax.experimental.pallas import tpu as pltpu
from jax.experimental.pallas import tpu_sc as plsc

K = 16
STRIP = 8 * K


def _oddeven_merge(lo, hi, r):
    step = r * 2
    if step < hi - lo:
        yield from _oddeven_merge(lo, hi, step)
        yield from _oddeven_merge(lo + r, hi, step)
        yield from ((i, i + r) for i in range(lo + r, hi - r, step))
    else:
        yield (lo, lo + r)


def _oddeven_sort_pairs(lo, hi):
    if hi - lo > 1:
        mid = lo + (hi - lo) // 2
        yield from _oddeven_sort_pairs(lo, mid)
        yield from _oddeven_sort_pairs(mid, hi)
        yield from _oddeven_merge(lo, hi - 1, 1)


_SORT16 = tuple(_oddeven_sort_pairs(0, K))


def _ce(p, i, j):
    hi = jnp.maximum(p[i], p[j])
    lo = jnp.minimum(p[i], p[j])
    p[i], p[j] = hi, lo


def _sort_desc(p):
    for i, j in _SORT16:
        _ce(p, i, j)
    return p


def _bitonic_merge_desc(p):
    j = K // 2
    while j >= 1:
        for base in range(0, K, 2 * j):
            for i in range(base, base + j):
                _ce(p, i, i + j)
        j //= 2
    return p


def _merge_sorted(acc, new):
    p = [jnp.maximum(acc[j], new[K - 1 - j]) for j in range(K)]
    return _bitonic_merge_desc(p)


def _strip_planes(v):
    return _sort_desc([v[j * 8:(j + 1) * 8] for j in range(K)])


def _topk_body(x_ref, o_ref):
    s = x_ref.shape[1]
    npairs = s // (2 * STRIP)

    first = _strip_planes(x_ref[0, 0:STRIP, :])
    second = _strip_planes(x_ref[0, STRIP:2 * STRIP, :])
    acc = _merge_sorted(first, second)

    def body(t, acc):
        v1 = x_ref[0, pl.ds(t * (2 * STRIP), STRIP), :]
        v2 = x_ref[0, pl.ds(t * (2 * STRIP) + STRIP, STRIP), :]
        m = _merge_sorted(_strip_planes(v1), _strip_planes(v2))
        return tuple(_merge_sorted(list(acc), m))

    acc = list(jax.lax.fori_loop(1, npairs, body, tuple(acc)))

    g = 8
    while g > 1:
        h = g // 2
        a = [q[:h] for q in acc]
        b = [q[h:] for q in acc]
        acc = _bitonic_merge_desc(
            [jnp.maximum(a[j], b[K - 1 - j]) for j in range(K)])
        g = h

    o_ref[0] = jnp.concatenate(acc, axis=0)


def _tc_topk(x, nb):
    b, s, f = x.shape
    return pl.pallas_call(
        _topk_body,
        grid=(nb,),
        in_specs=[pl.BlockSpec((1, s, f), lambda i: (i, 0, 0))],
        out_specs=pl.BlockSpec((1, K, f), lambda i: (i, 0, 0)),
        out_shape=jax.ShapeDtypeStruct((nb, K, f), x.dtype),
        compiler_params=pltpu.CompilerParams(
            dimension_semantics=("arbitrary",),
        ),
    )(x)


SC_NW = 32
SC_SBLK = 512
SC_L = 16


def _sc_process_block(buf, acc):
    def body(g, acc):
        p = _sort_desc([buf[g * K + j] for j in range(K)])
        return tuple(_merge_sorted(list(acc), p))

    return lax.fori_loop(0, SC_SBLK // K, body, acc)


def _sc_process_block_vsort(buf, acc):
    iot = lax.iota(jnp.int32, 16)

    def body(g, accs):
        rows = g * K + iot
        new = []
        for f in range(SC_L):
            col = jnp.full((16,), f, jnp.int32)
            v = plsc.load_gather(buf, [rows, col])
            nd = plsc.sort_key_val(v, v, descending=True)[0]
            c = jnp.maximum(accs[f], nd)
            new.append(plsc.sort_key_val(c, c)[0])
        return tuple(new)

    return lax.fori_loop(0, SC_SBLK // K, body, acc)


SC_UL = 16
SC_FI = 4


def _sc_topk(x, b0, nb):
    full_b, s, f = x.shape
    nfc = f // SC_UL
    units = nb * nfc
    nblocks = s // SC_SBLK
    mesh = plsc.VectorSubcoreMesh(core_axis_name="c", subcore_axis_name="s")

    @functools.partial(
        pl.kernel,
        mesh=mesh,
        out_type=jax.ShapeDtypeStruct((nb, K, f), jnp.float32),
        compiler_params=pltpu.CompilerParams(
            use_tc_tiling_on_sc=False, needs_layout_passes=False),
        scratch_types=[
            pltpu.VMEM((SC_SBLK, SC_UL), jnp.float32),
            pltpu.VMEM((SC_SBLK, SC_UL), jnp.float32),
            pltpu.VMEM((SC_UL, K), jnp.float32),
            pltpu.VMEM((K, SC_UL), jnp.float32),
            pltpu.SemaphoreType.DMA,
            pltpu.SemaphoreType.DMA,
        ],
    )
    def k(x_hbm, o_hbm, buf0, buf1, abuf, obuf, sem0, sem1):
        wid = lax.axis_index("s") * 2 + lax.axis_index("c")
        iot = lax.iota(jnp.int32, 16)

        ng_half = SC_SBLK // K // 2

        def process(buf, acc):
            def gbody(gp, acc):
                p = _sort_desc([buf[gp * K + j] for j in range(K)])
                acc = tuple(_merge_sorted(list(acc), p))
                rows = (ng_half + gp) * K + iot
                for ff in range(SC_L):
                    v = plsc.load_gather(
                        buf, [rows, jnp.full((16,), ff, jnp.int32)])
                    nd = plsc.sort_key_val(v, v, descending=True)[0]
                    c = jnp.maximum(abuf[ff], nd)
                    abuf[ff] = plsc.sort_key_val(c, c)[0]
                return acc

            return lax.fori_loop(0, ng_half, gbody, acc)

        for t in range(pl.cdiv(units, SC_NW)):
            u = wid + SC_NW * t

            @pl.when(u < units)
            def _():
                b = b0 + u // nfc
                f0 = (u % nfc) * SC_UL

                def src(blk):
                    return x_hbm.at[b, pl.ds(blk * SC_SBLK, SC_SBLK),
                                    pl.ds(f0, SC_UL)]

                pltpu.async_copy(src(0), buf0, sem0)
                pltpu.async_copy(src(1), buf1, sem1)

                ninf = jnp.full((K,), -jnp.inf, jnp.float32)
                for ff in range(SC_L):
                    abuf[ff] = ninf
                acc0 = tuple(ninf for _ in range(K))

                def body(i, acc, _src=src):
                    pltpu.make_async_copy(_src(0), buf0, sem0).wait()
                    acc = process(buf0, acc)

                    @pl.when(2 * i + 2 < nblocks)
                    def _():
                        pltpu.async_copy(_src(2 * i + 2), buf0, sem0)

                    pltpu.make_async_copy(_src(1), buf1, sem1).wait()
                    acc = process(buf1, acc)

                    @pl.when(2 * i + 3 < nblocks)
                    def _():
                        pltpu.async_copy(_src(2 * i + 3), buf1, sem1)

                    return acc

                acc = lax.fori_loop(0, nblocks // 2, body, acc0)

                vplanes = [
                    plsc.load_gather(
                        abuf, [iot, jnp.full((16,), 15 - j, jnp.int32)])
                    for j in range(K)
                ]
                acc = _merge_sorted(list(acc), vplanes)
                for j in range(K):
                    obuf[j] = acc[j]
                pltpu.sync_copy(obuf, o_hbm.at[u // nfc, :, pl.ds(f0, SC_UL)])

    return k(x)


TC_BATCHES = 48


@jax.jit
def kernel(x):
    b, s, f = x.shape
    n_tc = TC_BATCHES
    tc_out = _tc_topk(x, n_tc)
    sc_out = _sc_topk(x, n_tc, b - n_tc)
    return jnp.concatenate([tc_out, sc_out], axis=0)


# speedup vs baseline: 3.3759x; 3.3759x over previous
import functools

import jax
import jax.numpy as jnp
from jax import lax
from jax.experimental import pallas as pl
from jax.experimental.pallas import tpu as pltpu
from jax.experimental.pallas import tpu_sc as plsc

K = 16
STRIP = 8 * K


_SORT16 = (
    (0, 1), (2, 3), (4, 5), (6, 7), (8, 9), (10, 11), (12, 13), (14, 15),
    (0, 2), (1, 3), (4, 6), (5, 7), (8, 10), (9, 11), (12, 14), (13, 15),
    (0, 4), (1, 5), (2, 6), (3, 7), (8, 12), (9, 13), (10, 14), (11, 15),
    (0, 8), (1, 9), (2, 10), (3, 11), (4, 12), (5, 13), (6, 14), (7, 15),
    (5, 10), (6, 9), (3, 12), (13, 14), (7, 11), (1, 2), (4, 8),
    (1, 4), (7, 13), (2, 8), (11, 14), (5, 6), (9, 10),
    (2, 4), (11, 13), (3, 8), (7, 12),
    (6, 8), (10, 12), (3, 5), (7, 9),
    (3, 4), (5, 6), (7, 8), (9, 10), (11, 12),
    (6, 7), (8, 9),
)


def _ce(p, i, j):
    hi = jnp.maximum(p[i], p[j])
    lo = jnp.minimum(p[i], p[j])
    p[i], p[j] = hi, lo


def _sort_desc(p):
    for i, j in _SORT16:
        _ce(p, i, j)
    return p


def _bitonic_merge_desc(p):
    j = K // 2
    while j >= 1:
        for base in range(0, K, 2 * j):
            for i in range(base, base + j):
                _ce(p, i, i + j)
        j //= 2
    return p


def _merge_sorted(acc, new):
    p = [jnp.maximum(acc[j], new[K - 1 - j]) for j in range(K)]
    return _bitonic_merge_desc(p)


def _strip_planes(v):
    return _sort_desc([v[j * 8:(j + 1) * 8] for j in range(K)])


def _topk_body(x_ref, o_ref):
    s = x_ref.shape[1]
    npairs = s // (2 * STRIP)

    first = _strip_planes(x_ref[0, 0:STRIP, :])
    second = _strip_planes(x_ref[0, STRIP:2 * STRIP, :])
    acc = _merge_sorted(first, second)

    def body(t, acc):
        v1 = x_ref[0, pl.ds(t * (2 * STRIP), STRIP), :]
        v2 = x_ref[0, pl.ds(t * (2 * STRIP) + STRIP, STRIP), :]
        m = _merge_sorted(_strip_planes(v1), _strip_planes(v2))
        return tuple(_merge_sorted(list(acc), m))

    acc = list(jax.lax.fori_loop(1, npairs, body, tuple(acc)))

    g = 8
    while g > 1:
        h = g // 2
        a = [q[:h] for q in acc]
        b = [q[h:] for q in acc]
        acc = _bitonic_merge_desc(
            [jnp.maximum(a[j], b[K - 1 - j]) for j in range(K)])
        g = h

    o_ref[0] = jnp.concatenate(acc, axis=0)


def _tc_topk(x, nb):
    b, s, f = x.shape
    return pl.pallas_call(
        _topk_body,
        grid=(nb,),
        in_specs=[pl.BlockSpec((1, s, f), lambda i: (i, 0, 0))],
        out_specs=pl.BlockSpec((1, K, f), lambda i: (i, 0, 0)),
        out_shape=jax.ShapeDtypeStruct((nb, K, f), x.dtype),
        compiler_params=pltpu.CompilerParams(
            dimension_semantics=("arbitrary",),
        ),
    )(x)


SC_NW = 32
SC_SBLK = 512
SC_L = 16


SC_UL = 16


def _sc_topk(x, b0, nb):
    full_b, s, f = x.shape
    nfc = f // SC_UL
    units = nb * nfc
    nblocks = s // SC_SBLK
    mesh = plsc.VectorSubcoreMesh(core_axis_name="c", subcore_axis_name="s")

    @functools.partial(
        pl.kernel,
        mesh=mesh,
        out_type=jax.ShapeDtypeStruct((nb, K, f), jnp.float32),
        compiler_params=pltpu.CompilerParams(
            use_tc_tiling_on_sc=False, needs_layout_passes=False),
        scratch_types=[
            pltpu.VMEM((SC_SBLK, SC_UL), jnp.float32),
            pltpu.VMEM((SC_SBLK, SC_UL), jnp.float32),
            pltpu.VMEM((SC_UL, K), jnp.float32),
            pltpu.VMEM((K, SC_UL), jnp.float32),
            pltpu.SemaphoreType.DMA,
            pltpu.SemaphoreType.DMA,
        ],
    )
    def k(x_hbm, o_hbm, buf0, buf1, abuf, obuf, sem0, sem1):
        wid = lax.axis_index("s") * 2 + lax.axis_index("c")
        iot = lax.iota(jnp.int32, 16)

        def process(buf):
            for c in range(SC_UL // SC_L):
                acc = tuple(abuf[c * K + j] for j in range(K))

                def gbody(g, acc, _c=c):
                    p = _sort_desc(
                        [buf[g * K + j, pl.ds(_c * SC_L, SC_L)]
                         for j in range(K)])
                    return tuple(_merge_sorted(list(acc), p))

                acc = lax.fori_loop(0, SC_SBLK // K, gbody, acc)
                for j in range(K):
                    abuf[c * K + j] = acc[j]

        for t in range(pl.cdiv(units, SC_NW)):
            u = wid + SC_NW * t

            @pl.when(u < units)
            def _():
                b = b0 + u // nfc
                f0 = (u % nfc) * SC_UL

                def src(blk):
                    return x_hbm.at[b, pl.ds(blk * SC_SBLK, SC_SBLK),
                                    pl.ds(f0, SC_UL)]

                pltpu.async_copy(src(0), buf0, sem0)
                pltpu.async_copy(src(1), buf1, sem1)

                ninf = jnp.full((K,), -jnp.inf, jnp.float32)
                for ff in range(SC_UL):
                    abuf[ff] = ninf

                def body(i, carry, _src=src):
                    pltpu.make_async_copy(_src(0), buf0, sem0).wait()
                    process(buf0)

                    @pl.when(2 * i + 2 < nblocks)
                    def _():
                        pltpu.async_copy(_src(2 * i + 2), buf0, sem0)

                    pltpu.make_async_copy(_src(1), buf1, sem1).wait()
                    process(buf1)

                    @pl.when(2 * i + 3 < nblocks)
                    def _():
                        pltpu.async_copy(_src(2 * i + 3), buf1, sem1)

                    return carry

                lax.fori_loop(0, nblocks // 2, body, 0)

                for c in range(SC_UL // SC_L):
                    for j in range(K):
                        obuf[j, pl.ds(c * SC_L, SC_L)] = abuf[c * K + j]
                pltpu.sync_copy(obuf, o_hbm.at[u // nfc, :, pl.ds(f0, SC_UL)])

    return k(x)


TC_BATCHES = 48


@jax.jit
def kernel(x):
    b, s, f = x.shape
    n_tc = TC_BATCHES
    tc_out = _tc_topk(x, n_tc)
    sc_out = _sc_topk(x, n_tc, b - n_tc)
    return jnp.concatenate([tc_out, sc_out], axis=0)
